# baseline (speedup 1.0000x reference)
import jax
import jax.numpy as jnp
from jax.experimental import pallas as pl
from jax.experimental.pallas import tpu as pltpu

B, T, F = 16, 2048, 1024


def _body(x_ref, o_ref):
    b = pl.program_id(0)
    x = x_ref[0]
    m = jnp.max(jnp.abs(x), axis=1)
    c = jnp.sum((m > 0.0).astype(jnp.int32))
    t = jnp.maximum(c - 1, 0)
    o_ref[pl.ds(b, 1), :] = x_ref[0, pl.ds(t, 1), :]


_fused = pl.pallas_call(
    _body,
    grid=(B,),
    in_specs=[pl.BlockSpec((1, T, F), lambda b: (b, 0, 0))],
    out_specs=pl.BlockSpec((B, F), lambda b: (0, 0)),
    out_shape=jax.ShapeDtypeStruct((B, F), jnp.float32),
    compiler_params=pltpu.CompilerParams(
        dimension_semantics=("arbitrary",),
    ),
)


def kernel(inputs):
    return _fused(inputs)
